# baseline (speedup 1.0000x reference)
import functools

import jax
import jax.numpy as jnp
from jax import lax
from jax.experimental import pallas as pl
from jax.experimental.pallas import tpu as pltpu

_LN_EPS = 1e-5


def _fused_kernel(x_ref, w_ref, b_ref, o_ref, s_ref, *, inv_e, tn):
    phg = pl.program_id(1)
    v = x_ref[...].astype(jnp.bfloat16)
    v = v.reshape(v.shape[0], 3, 14, 8, 14, 16)
    for c in range(3):
        for hp in range(14):
            for ph8 in range(8):
                s_ref[:, phg, hp * 16:hp * 16 + 14,
                      c * 128 + ph8 * 16:c * 128 + ph8 * 16 + 16] = \
                    v[:, c, hp, ph8]

    @pl.when(phg == 1)
    def _epilogue():
        params = b_ref[...]
        for t in range(tn):
            acc = (jnp.dot(s_ref[t, 0], w_ref[0],
                           preferred_element_type=jnp.float32)
                   + jnp.dot(s_ref[t, 1], w_ref[1],
                             preferred_element_type=jnp.float32))
            acc = acc + params[0:1]
            mean = jnp.sum(acc, axis=-1, keepdims=True) * inv_e
            sumsq = jnp.sum(acc * acc, axis=-1, keepdims=True) * inv_e
            var = jnp.maximum(sumsq - mean * mean, 0.0)
            normed = (acc - mean) * lax.rsqrt(var + _LN_EPS)
            out = normed * params[1:2] + params[2:3]
            out_t = jnp.transpose(out, (1, 0)).astype(o_ref.dtype)
            for hp in range(14):
                o_ref[t, :, hp * 14:(hp + 1) * 14] = \
                    out_t[:, hp * 16:hp * 16 + 14]


def kernel(x, conv_w, conv_b, ln_g, ln_b):
    N, C, H, W = x.shape
    E = conv_w.shape[0]
    P = 16
    Hp, Wp = H // P, W // P
    HW = Hp * Wp
    K = C * P * P

    w_r = conv_w.reshape(E, C, 2, 8, P)
    w_r = jnp.transpose(w_r, (2, 1, 3, 4, 0))
    w_r = w_r.reshape(2, K // 2, E).astype(jnp.bfloat16)
    params = jnp.stack([conv_b, ln_g, ln_b]).astype(jnp.float32)
    params = jnp.pad(params, ((0, 8 - 3), (0, 0)))

    tn = 4
    grid = (N // tn, 2)
    cost = pl.CostEstimate(
        flops=2 * N * HW * K * E,
        transcendentals=N * HW,
        bytes_accessed=(N * C * H * W * 4 + K * E * 2 + 8 * E * 4
                        + N * E * HW * 4))

    out = pl.pallas_call(
        functools.partial(_fused_kernel, inv_e=1.0 / E, tn=tn),
        out_shape=jax.ShapeDtypeStruct((N, E, HW), x.dtype),
        grid=grid,
        in_specs=[
            pl.BlockSpec((tn, C, Hp, 1, 8, Wp, P),
                         lambda i, g: (i, 0, 0, g, 0, 0, 0)),
            pl.BlockSpec((2, K // 2, E), lambda i, g: (0, 0, 0)),
            pl.BlockSpec((8, E), lambda i, g: (0, 0)),
        ],
        out_specs=pl.BlockSpec((tn, E, HW), lambda i, g: (i, 0, 0)),
        scratch_shapes=[pltpu.VMEM((tn, 2, Hp * P, K // 2), jnp.bfloat16)],
        compiler_params=pltpu.CompilerParams(
            dimension_semantics=("parallel", "arbitrary"),
            vmem_limit_bytes=60 * 1024 * 1024),
        cost_estimate=cost,
    )(x.reshape(N, C, Hp, 2, 8, Wp, P), w_r, params)

    return out.reshape(N, E, Hp, Wp)


# speedup vs baseline: 2.6314x; 2.6314x over previous
import functools

import jax
import jax.numpy as jnp
from jax import lax
from jax.experimental import pallas as pl
from jax.experimental.pallas import tpu as pltpu

_LN_EPS = 1e-5


def _fused_kernel(p_ref, w_ref, b_ref, o_ref, *, inv_e, tn):
    params = b_ref[...]
    for t in range(tn):
        acc = jnp.dot(p_ref[t], w_ref[...],
                      preferred_element_type=jnp.float32)
        acc = acc + params[0:1]
        mean = jnp.sum(acc, axis=-1, keepdims=True) * inv_e
        sumsq = jnp.sum(acc * acc, axis=-1, keepdims=True) * inv_e
        var = jnp.maximum(sumsq - mean * mean, 0.0)
        normed = (acc - mean) * lax.rsqrt(var + _LN_EPS)
        out = normed * params[1:2] + params[2:3]
        o_ref[t] = jnp.transpose(out, (1, 0)).astype(o_ref.dtype)


def kernel(x, conv_w, conv_b, ln_g, ln_b):
    N, C, H, W = x.shape
    E = conv_w.shape[0]
    P = 16
    Hp, Wp = H // P, W // P
    HW = Hp * Wp
    K = C * P * P
    compute_dtype = jnp.bfloat16

    patches = x.astype(compute_dtype).reshape(N, C, Hp, P, Wp, P)
    patches = jnp.transpose(patches, (0, 2, 4, 1, 3, 5)).reshape(N, HW, K)

    w2d = conv_w.reshape(E, K).T.astype(compute_dtype)
    params = jnp.stack([conv_b, ln_g, ln_b]).astype(jnp.float32)
    params = jnp.pad(params, ((0, 8 - 3), (0, 0)))

    tn = 4
    grid = (N // tn,)
    cost = pl.CostEstimate(
        flops=2 * N * HW * K * E,
        transcendentals=N * HW,
        bytes_accessed=(N * HW * K * 2 + K * E * 2 + 8 * E * 4
                        + N * E * HW * 4))

    out = pl.pallas_call(
        functools.partial(_fused_kernel, inv_e=1.0 / E, tn=tn),
        out_shape=jax.ShapeDtypeStruct((N, E, HW), x.dtype),
        grid=grid,
        in_specs=[
            pl.BlockSpec((tn, HW, K), lambda i: (i, 0, 0)),
            pl.BlockSpec((K, E), lambda i: (0, 0)),
            pl.BlockSpec((8, E), lambda i: (0, 0)),
        ],
        out_specs=pl.BlockSpec((tn, E, HW), lambda i: (i, 0, 0)),
        compiler_params=pltpu.CompilerParams(
            dimension_semantics=("parallel",),
            vmem_limit_bytes=96 * 1024 * 1024),
        cost_estimate=cost,
    )(patches, w2d, params)

    return out.reshape(N, E, Hp, Wp)
